# baseline (speedup 1.0000x reference)
import jax
import jax.numpy as jnp
from jax.experimental import pallas as pl
from jax.experimental.pallas import tpu as pltpu

_H = 16
_HD = 128
_K = 2048
_OUT = 3 * 2048
_BB = 512


def _cast_body(w_ref, o_ref):
    o_ref[...] = w_ref[...].astype(jnp.bfloat16)


def _qkv_body(x_ref, w_ref, q_ref, k_ref, v_ref):
    xv = x_ref[...].astype(jnp.bfloat16)
    acc = jax.lax.dot_general(
        xv, w_ref[...], (((1,), (1,)), ((), ())),
        preferred_element_type=jnp.float32,
    )
    for i, ref in enumerate((q_ref, k_ref, v_ref)):
        for h in range(_H):
            col = i * 2048 + h * _HD
            ref[h] = acc[:, col:col + _HD]


@jax.jit
def kernel(x, W_qkv):
    batch = x.shape[0]
    wb = pl.pallas_call(
        _cast_body,
        grid=(8,),
        in_specs=[pl.BlockSpec((_OUT // 8, _K), lambda c: (c, 0))],
        out_specs=pl.BlockSpec((_OUT // 8, _K), lambda c: (c, 0)),
        out_shape=jax.ShapeDtypeStruct((_OUT, _K), jnp.bfloat16),
    )(W_qkv)
    out_sd = jax.ShapeDtypeStruct((_H, batch, _HD), jnp.float32)
    q, k, v = pl.pallas_call(
        _qkv_body,
        grid=(batch // _BB,),
        in_specs=[
            pl.BlockSpec((_BB, _K), lambda b: (b, 0)),
            pl.BlockSpec((_OUT, _K), lambda b: (0, 0)),
        ],
        out_specs=[
            pl.BlockSpec((_H, _BB, _HD), lambda b: (0, b, 0)),
            pl.BlockSpec((_H, _BB, _HD), lambda b: (0, b, 0)),
            pl.BlockSpec((_H, _BB, _HD), lambda b: (0, b, 0)),
        ],
        out_shape=(out_sd, out_sd, out_sd),
        compiler_params=pltpu.CompilerParams(
            dimension_semantics=("parallel",),
        ),
    )(x, wb)
    return q, k, v


# speedup vs baseline: 1.0691x; 1.0691x over previous
import jax
import jax.numpy as jnp
from jax.experimental import pallas as pl
from jax.experimental.pallas import tpu as pltpu

_H = 16
_HD = 128
_K = 2048
_OUT = 3 * 2048
_BB = 512
_NCH = 24
_CR = _OUT // _NCH
_NM = 4096 // _BB


def _qkv_body(x_ref, w_ref, q_ref, k_ref, v_ref, wb_ref):
    t = pl.program_id(0)

    @pl.when(t < _NCH)
    def _cast():
        wb_ref[pl.ds(t * _CR, _CR), :] = w_ref[0].astype(jnp.bfloat16)

    @pl.when(t >= _NCH)
    def _matmul():
        xv = x_ref[...].astype(jnp.bfloat16)
        acc = jax.lax.dot_general(
            xv, wb_ref[...], (((1,), (1,)), ((), ())),
            preferred_element_type=jnp.float32,
        )
        for i, ref in enumerate((q_ref, k_ref, v_ref)):
            for h in range(_H):
                col = i * 2048 + h * _HD
                ref[h] = acc[:, col:col + _HD]


def _clamp_m(t):
    return jnp.clip(t - _NCH, 0, _NM - 1)


@jax.jit
def kernel(x, W_qkv):
    batch = x.shape[0]
    w4 = W_qkv.reshape(_NCH, _CR, _K)
    out_sd = jax.ShapeDtypeStruct((_H, batch, _HD), jnp.float32)
    q, k, v = pl.pallas_call(
        _qkv_body,
        grid=(_NCH + batch // _BB,),
        in_specs=[
            pl.BlockSpec((_BB, _K), lambda t: (_clamp_m(t), 0)),
            pl.BlockSpec((1, _CR, _K), lambda t: (jnp.clip(t, 0, _NCH - 1), 0, 0)),
        ],
        out_specs=[
            pl.BlockSpec((_H, _BB, _HD), lambda t: (0, _clamp_m(t), 0)),
            pl.BlockSpec((_H, _BB, _HD), lambda t: (0, _clamp_m(t), 0)),
            pl.BlockSpec((_H, _BB, _HD), lambda t: (0, _clamp_m(t), 0)),
        ],
        out_shape=(out_sd, out_sd, out_sd),
        scratch_shapes=[pltpu.VMEM((_OUT, _K), jnp.bfloat16)],
        compiler_params=pltpu.CompilerParams(
            vmem_limit_bytes=63 * 1024 * 1024,
        ),
    )(x, w4)
    return q, k, v


# speedup vs baseline: 1.1140x; 1.0421x over previous
import jax
import jax.numpy as jnp
from jax.experimental import pallas as pl
from jax.experimental.pallas import tpu as pltpu

_H = 16
_HD = 128
_K = 2048
_OUT = 3 * 2048
_BB = 512
_NM = 4096 // _BB
_NCH = 24
_CR = _OUT // _NCH


def _qkv_body(x_ref, w_ref, q_ref, k_ref, v_ref, wb_ref):
    t = pl.program_id(0)

    @pl.when(t < _NCH)
    def _cast():
        wb_ref[pl.ds(t * _CR, _CR), :] = w_ref[0].astype(jnp.bfloat16)

    for i, ref in enumerate((q_ref, k_ref, v_ref)):
        lo = _NM * (i + 1)

        @pl.when((t >= lo) & (t < lo + _NM))
        def _matmul(ref=ref, i=i):
            xv = x_ref[...].astype(jnp.bfloat16)
            acc = jax.lax.dot_general(
                xv, wb_ref[i * 2048:(i + 1) * 2048, :],
                (((1,), (1,)), ((), ())),
                preferred_element_type=jnp.float32,
            )
            for h in range(_H):
                ref[h] = acc[:, h * _HD:(h + 1) * _HD]


@jax.jit
def kernel(x, W_qkv):
    batch = x.shape[0]
    w24 = W_qkv.reshape(_NCH, _CR, _K)
    out_sd = jax.ShapeDtypeStruct((_H, batch, _HD), jnp.float32)
    q, k, v = pl.pallas_call(
        _qkv_body,
        grid=(4 * _NM,),
        in_specs=[
            pl.BlockSpec(
                (_BB, _K),
                lambda t: (jnp.where(t < _NM, 0, jnp.remainder(t, _NM)), 0),
            ),
            pl.BlockSpec(
                (1, _CR, _K),
                lambda t: (jnp.clip(t, 0, _NCH - 1), 0, 0),
            ),
        ],
        out_specs=[
            pl.BlockSpec((_H, _BB, _HD),
                         lambda t: (0, jnp.clip(t - _NM, 0, _NM - 1), 0)),
            pl.BlockSpec((_H, _BB, _HD),
                         lambda t: (0, jnp.clip(t - 2 * _NM, 0, _NM - 1), 0)),
            pl.BlockSpec((_H, _BB, _HD),
                         lambda t: (0, jnp.clip(t - 3 * _NM, 0, _NM - 1), 0)),
        ],
        out_shape=(out_sd, out_sd, out_sd),
        scratch_shapes=[pltpu.VMEM((_OUT, _K), jnp.bfloat16)],
        compiler_params=pltpu.CompilerParams(
            vmem_limit_bytes=63 * 1024 * 1024,
        ),
    )(x, w24)
    return q, k, v


# speedup vs baseline: 1.1381x; 1.0216x over previous
import jax
import jax.numpy as jnp
from jax.experimental import pallas as pl
from jax.experimental.pallas import tpu as pltpu

_H = 16
_HD = 128
_K = 2048
_OUT = 3 * 2048
_BB = 512
_NM = 4096 // _BB
_NCH = 16
_CR = _OUT // _NCH
_HALF = _OUT // 2


def _qkv_body(x_ref, w_ref, q_ref, k_ref, v_ref, wb_ref):
    t = pl.program_id(0)

    @pl.when(t < _NCH)
    def _cast():
        wb_ref[pl.ds(t * _CR, _CR), :] = w_ref[0].astype(jnp.bfloat16)

    @pl.when((t >= _NM) & (t < 2 * _NM))
    def _phase_a():
        xv = x_ref[...].astype(jnp.bfloat16)
        acc = jax.lax.dot_general(
            xv, wb_ref[0:_HALF, :], (((1,), (1,)), ((), ())),
            preferred_element_type=jnp.float32,
        )
        for h in range(_H):
            q_ref[h] = acc[:, h * _HD:(h + 1) * _HD]
        for j in range(8):
            k_ref[j] = acc[:, 2048 + j * _HD:2048 + (j + 1) * _HD]

    @pl.when(t >= 2 * _NM)
    def _phase_b():
        xv = x_ref[...].astype(jnp.bfloat16)
        acc = jax.lax.dot_general(
            xv, wb_ref[_HALF:_OUT, :], (((1,), (1,)), ((), ())),
            preferred_element_type=jnp.float32,
        )
        for j in range(8):
            k_ref[j] = acc[:, j * _HD:(j + 1) * _HD]
        for h in range(_H):
            v_ref[h] = acc[:, 1024 + h * _HD:1024 + (h + 1) * _HD]


@jax.jit
def kernel(x, W_qkv):
    batch = x.shape[0]
    w16 = W_qkv.reshape(_NCH, _CR, _K)
    out_sd = jax.ShapeDtypeStruct((_H, batch, _HD), jnp.float32)
    q, k, v = pl.pallas_call(
        _qkv_body,
        grid=(3 * _NM,),
        in_specs=[
            pl.BlockSpec(
                (_BB, _K),
                lambda t: (jnp.where(t < _NM, 0,
                                     jnp.remainder(t - _NM, _NM)), 0),
            ),
            pl.BlockSpec(
                (1, _CR, _K),
                lambda t: (jnp.clip(t, 0, _NCH - 1), 0, 0),
            ),
        ],
        out_specs=[
            pl.BlockSpec((_H, _BB, _HD),
                         lambda t: (0, jnp.clip(t - _NM, 0, _NM - 1), 0)),
            pl.BlockSpec(
                (8, _BB, _HD),
                lambda t: (jnp.where(t < 2 * _NM, 0, 1),
                           jnp.where(t < 2 * _NM,
                                     jnp.clip(t - _NM, 0, _NM - 1),
                                     jnp.clip(t - 2 * _NM, 0, _NM - 1)),
                           0),
            ),
            pl.BlockSpec((_H, _BB, _HD),
                         lambda t: (0, jnp.clip(t - 2 * _NM, 0, _NM - 1), 0)),
        ],
        out_shape=(out_sd, out_sd, out_sd),
        scratch_shapes=[pltpu.VMEM((_OUT, _K), jnp.bfloat16)],
        compiler_params=pltpu.CompilerParams(
            vmem_limit_bytes=63 * 1024 * 1024,
        ),
    )(x, w16)
    return q, k, v
